# baseline (speedup 1.0000x reference)
import numpy as np
import jax
import jax.numpy as jnp
from jax import lax
from jax.experimental import pallas as pl
from jax.experimental.pallas import tpu as pltpu
from jax.experimental.pallas import tpu_sc as plsc

B = 4096
D_IN = 13
NT = 26
V = 100000
ED = 64
NP1 = NT + 1

_NC = 2
_NS = 16
_NW = _NC * _NS
_ROWS = B * NT
_RPW = _ROWS // _NW
_CHUNK = 128
_NCHUNK = _RPW // _CHUNK
_HALF = _NCHUNK // 2

_BBLK = 256
_XW = 64 + NP1 * NP1


_GRP = 3
_NGRP = (_NCHUNK + _GRP - 1) // _GRP
_PW = 2 * ED


def _sc_gather_body(tpair, idx_hbm, out_hbm, idx_v, rows_a, rows_b, sem):
    wid = lax.axis_index("s") * _NC + lax.axis_index("c")
    base = wid * _RPW
    pltpu.sync_copy(idx_hbm.at[wid], idx_v)
    bufs = (rows_a, rows_b)
    pend = []
    for g in range(_NGRP):
        buf = bufs[g % 2]
        nch = min(_GRP, _NCHUNK - g * _GRP)
        cps = [
            pltpu.async_copy(
                tpair.at[idx_v.at[g * _GRP + j]],
                buf.at[pl.ds(j * _CHUNK, _CHUNK)],
                sem,
            )
            for j in range(nch)
        ]
        if pend:
            pg, pbuf, pcps = pend.pop()
            for cp in pcps:
                cp.wait()
            pn = min(_GRP, _NCHUNK - pg * _GRP) * _CHUNK
            pltpu.sync_copy(
                pbuf.at[pl.ds(0, pn)],
                out_hbm.at[pl.ds(base + pg * _GRP * _CHUNK, pn)],
            )
        pend.append((g, buf, cps))
    pg, pbuf, pcps = pend.pop()
    for cp in pcps:
        cp.wait()
    pn = min(_GRP, _NCHUNK - pg * _GRP) * _CHUNK
    pltpu.sync_copy(
        pbuf.at[pl.ds(0, pn)],
        out_hbm.at[pl.ds(base + pg * _GRP * _CHUNK, pn)],
    )


_SC_CACHE = {}


def _sc_gather(tpair, flat_idx):
    if "k" not in _SC_CACHE:
        _SC_CACHE["k"] = pl.kernel(
            _sc_gather_body,
            out_type=jax.ShapeDtypeStruct((_ROWS, _PW), jnp.float32),
            mesh=plsc.VectorSubcoreMesh(core_axis_name="c", subcore_axis_name="s"),
            scratch_types=[
                pltpu.VMEM((_NCHUNK, _CHUNK), jnp.int32),
                pltpu.VMEM((_GRP * _CHUNK, _PW), jnp.float32),
                pltpu.VMEM((_GRP * _CHUNK, _PW), jnp.float32),
                pltpu.SemaphoreType.DMA,
            ],
        )
    return _SC_CACHE["k"](tpair, flat_idx)


def _tc_body(dense_ref, pooled_ref, par_ref, wd0, bd0, wd1, bd1, wd2, bd2,
             w0f, bo0, wo1, bo1, wo2, bo2, wo3, bo3, out_ref):
    f32 = jnp.float32
    h = dense_ref[...]
    h = jnp.maximum(jnp.dot(h, wd0[...], preferred_element_type=f32) + bd0[...], 0.0)
    h = jnp.maximum(jnp.dot(h, wd1[...], preferred_element_type=f32) + bd1[...], 0.0)
    h = jnp.maximum(jnp.dot(h, wd2[...], preferred_element_type=f32) + bd2[...], 0.0)
    pp = pooled_ref[...]
    pooled = jnp.where(par_ref[...][:, :, None] == 1,
                       pp[:, :, ED:], pp[:, :, :ED])
    embs = jnp.concatenate([h[:, None, :], pooled], axis=1)
    cols = [h]
    for n_i in range(NP1):
        a = embs[:, n_i:n_i + 1, :]
        cols.append(jnp.sum(embs * a, axis=2))
    x = jnp.concatenate(cols, axis=1)
    x = jnp.maximum(jnp.dot(x, w0f[...], preferred_element_type=f32) + bo0[...], 0.0)
    x = jnp.maximum(jnp.dot(x, wo1[...], preferred_element_type=f32) + bo1[...], 0.0)
    x = jnp.maximum(jnp.dot(x, wo2[...], preferred_element_type=f32) + bo2[...], 0.0)
    out_ref[...] = jnp.dot(x, wo3[...], preferred_element_type=f32) + bo3[...]


_SEL = np.full((NP1 * NP1,), 442, dtype=np.int32)
_p = 0
for _n in range(NP1):
    for _m in range(_n, NP1):
        _SEL[_n * NP1 + _m] = 64 + _p
        _p += 1


def _rep(shape):
    nd = len(shape)
    return pl.BlockSpec(shape, lambda i, _nd=nd: (0,) * _nd)


def _tc_forward(dense, pooled, parity, wd0, bd0, wd1, bd1, wd2, bd2,
                w0f, bo0, wo1, bo1, wo2, bo2, wo3, bo3):
    grid = (B // _BBLK,)
    in_specs = [
        pl.BlockSpec((_BBLK, D_IN), lambda i: (i, 0)),
        pl.BlockSpec((_BBLK, NT, _PW), lambda i: (i, 0, 0)),
        pl.BlockSpec((_BBLK, NT), lambda i: (i, 0)),
        _rep(wd0.shape), _rep(bd0.shape), _rep(wd1.shape), _rep(bd1.shape),
        _rep(wd2.shape), _rep(bd2.shape), _rep(w0f.shape), _rep(bo0.shape),
        _rep(wo1.shape), _rep(bo1.shape), _rep(wo2.shape), _rep(bo2.shape),
        _rep(wo3.shape), _rep(bo3.shape),
    ]
    return pl.pallas_call(
        _tc_body,
        grid=grid,
        in_specs=in_specs,
        out_specs=pl.BlockSpec((_BBLK, 1), lambda i: (i, 0)),
        out_shape=jax.ShapeDtypeStruct((B, 1), jnp.float32),
    )(dense, pooled, parity, wd0, bd0, wd1, bd1, wd2, bd2,
      w0f, bo0, wo1, bo1, wo2, bo2, wo3, bo3)


def kernel(dense, sparse_indices, sparse_offsets, tables,
           w_d0, b_d0, w_d1, b_d1, w_d2, b_d2,
           w_o0, b_o0, w_o1, b_o1, w_o2, b_o2, w_o3, b_o3):
    del sparse_offsets
    tpair = tables.reshape(NT * V // 2, _PW)
    offs = jnp.arange(NT, dtype=jnp.int32) * (V // 2)
    pair_idx = ((sparse_indices >> 1).T + offs[None, :]).reshape(_NW, _NCHUNK, _CHUNK)
    parity = (sparse_indices & 1).T
    pooled = _sc_gather(tpair, pair_idx).reshape(B, NT, _PW)

    w_cat = jnp.concatenate([w_o0, jnp.zeros((1, w_o0.shape[1]), w_o0.dtype)], axis=0)
    w0f = jnp.concatenate([w_o0[:64], jnp.take(w_cat, _SEL, axis=0)], axis=0)

    return _tc_forward(
        dense, pooled, parity,
        w_d0, b_d0[None, :], w_d1, b_d1[None, :], w_d2, b_d2[None, :],
        w0f, b_o0[None, :], w_o1, b_o1[None, :], w_o2, b_o2[None, :],
        w_o3, b_o3[None, :],
    )


# speedup vs baseline: 1.2725x; 1.2725x over previous
import numpy as np
import jax
import jax.numpy as jnp
from jax import lax
from jax.experimental import pallas as pl
from jax.experimental.pallas import tpu as pltpu
from jax.experimental.pallas import tpu_sc as plsc

B = 4096
D_IN = 13
NT = 26
V = 100000
ED = 64
NP1 = NT + 1

_NC = 2
_NS = 16
_NW = _NC * _NS
_ROWS = B * NT
_RPW = _ROWS // _NW
_CHUNK = 128
_NCHUNK = _RPW // _CHUNK
_HALF = _NCHUNK // 2

_BBLK = 256
_XW = 64 + NP1 * NP1


_GRP = 3
_NGRP = (_NCHUNK + _GRP - 1) // _GRP
_PW = 2 * ED


def _sc_gather_body(tpair, idx_hbm, out_hbm, idx_v, rows_a, rows_b, sem):
    wid = lax.axis_index("s") * _NC + lax.axis_index("c")
    base = wid * _RPW
    pltpu.sync_copy(idx_hbm.at[wid], idx_v)
    bufs = (rows_a, rows_b)
    pend = []
    for g in range(_NGRP):
        buf = bufs[g % 2]
        nch = min(_GRP, _NCHUNK - g * _GRP)
        cps = [
            pltpu.async_copy(
                tpair.at[idx_v.at[g * _GRP + j]],
                buf.at[pl.ds(j * _CHUNK, _CHUNK)],
                sem,
            )
            for j in range(nch)
        ]
        if pend:
            pg, pbuf, pcps = pend.pop()
            for cp in pcps:
                cp.wait()
            pn = min(_GRP, _NCHUNK - pg * _GRP) * _CHUNK
            pltpu.sync_copy(
                pbuf.at[pl.ds(0, pn)],
                out_hbm.at[pl.ds(base + pg * _GRP * _CHUNK, pn)],
            )
        pend.append((g, buf, cps))
    pg, pbuf, pcps = pend.pop()
    for cp in pcps:
        cp.wait()
    pn = min(_GRP, _NCHUNK - pg * _GRP) * _CHUNK
    pltpu.sync_copy(
        pbuf.at[pl.ds(0, pn)],
        out_hbm.at[pl.ds(base + pg * _GRP * _CHUNK, pn)],
    )


_SC_CACHE = {}


def _sc_gather(tpair, flat_idx):
    if "k" not in _SC_CACHE:
        _SC_CACHE["k"] = pl.kernel(
            _sc_gather_body,
            out_type=jax.ShapeDtypeStruct((_ROWS, _PW), jnp.float32),
            mesh=plsc.VectorSubcoreMesh(core_axis_name="c", subcore_axis_name="s"),
            scratch_types=[
                pltpu.VMEM((_NCHUNK, _CHUNK), jnp.int32),
                pltpu.VMEM((_GRP * _CHUNK, _PW), jnp.float32),
                pltpu.VMEM((_GRP * _CHUNK, _PW), jnp.float32),
                pltpu.SemaphoreType.DMA,
            ],
        )
    return _SC_CACHE["k"](tpair, flat_idx)


def _tc_body(dense_ref, pooled_ref, par_ref, wd0, bd0, wd1, bd1, wd2, bd2,
             w0f, bo0, wo1, bo1, wo2, bo2, wo3, bo3, out_ref):
    f32 = jnp.float32
    h = dense_ref[...]
    h = jnp.maximum(jnp.dot(h, wd0[...], preferred_element_type=f32) + bd0[...], 0.0)
    h = jnp.maximum(jnp.dot(h, wd1[...], preferred_element_type=f32) + bd1[...], 0.0)
    h = jnp.maximum(jnp.dot(h, wd2[...], preferred_element_type=f32) + bd2[...], 0.0)
    pp = pooled_ref[...]
    pooled = jnp.where(par_ref[...][:, :, None] == 1,
                       pp[:, :, ED:], pp[:, :, :ED])
    embs = jnp.concatenate([h[:, None, :], pooled], axis=1)
    inter = jax.lax.dot_general(
        embs, embs, (((2,), (2,)), ((0,), (0,))),
        preferred_element_type=f32)
    x = jnp.concatenate([h, inter.reshape(_BBLK, NP1 * NP1)], axis=1)
    x = jnp.maximum(jnp.dot(x, w0f[...], preferred_element_type=f32) + bo0[...], 0.0)
    x = jnp.maximum(jnp.dot(x, wo1[...], preferred_element_type=f32) + bo1[...], 0.0)
    x = jnp.maximum(jnp.dot(x, wo2[...], preferred_element_type=f32) + bo2[...], 0.0)
    out_ref[...] = jnp.dot(x, wo3[...], preferred_element_type=f32) + bo3[...]


_SEL = np.full((NP1 * NP1,), 442, dtype=np.int32)
_p = 0
for _n in range(NP1):
    for _m in range(_n, NP1):
        _SEL[_n * NP1 + _m] = 64 + _p
        _p += 1


def _rep(shape):
    nd = len(shape)
    return pl.BlockSpec(shape, lambda i, _nd=nd: (0,) * _nd)


def _tc_forward(dense, pooled, parity, wd0, bd0, wd1, bd1, wd2, bd2,
                w0f, bo0, wo1, bo1, wo2, bo2, wo3, bo3):
    grid = (B // _BBLK,)
    in_specs = [
        pl.BlockSpec((_BBLK, D_IN), lambda i: (i, 0)),
        pl.BlockSpec((_BBLK, NT, _PW), lambda i: (i, 0, 0)),
        pl.BlockSpec((_BBLK, NT), lambda i: (i, 0)),
        _rep(wd0.shape), _rep(bd0.shape), _rep(wd1.shape), _rep(bd1.shape),
        _rep(wd2.shape), _rep(bd2.shape), _rep(w0f.shape), _rep(bo0.shape),
        _rep(wo1.shape), _rep(bo1.shape), _rep(wo2.shape), _rep(bo2.shape),
        _rep(wo3.shape), _rep(bo3.shape),
    ]
    return pl.pallas_call(
        _tc_body,
        grid=grid,
        in_specs=in_specs,
        out_specs=pl.BlockSpec((_BBLK, 1), lambda i: (i, 0)),
        out_shape=jax.ShapeDtypeStruct((B, 1), jnp.float32),
    )(dense, pooled, parity, wd0, bd0, wd1, bd1, wd2, bd2,
      w0f, bo0, wo1, bo1, wo2, bo2, wo3, bo3)


def kernel(dense, sparse_indices, sparse_offsets, tables,
           w_d0, b_d0, w_d1, b_d1, w_d2, b_d2,
           w_o0, b_o0, w_o1, b_o1, w_o2, b_o2, w_o3, b_o3):
    del sparse_offsets
    tpair = tables.reshape(NT * V // 2, _PW)
    offs = jnp.arange(NT, dtype=jnp.int32) * (V // 2)
    pair_idx = ((sparse_indices >> 1).T + offs[None, :]).reshape(_NW, _NCHUNK, _CHUNK)
    parity = (sparse_indices & 1).T
    pooled = _sc_gather(tpair, pair_idx).reshape(B, NT, _PW)

    w_cat = jnp.concatenate([w_o0, jnp.zeros((1, w_o0.shape[1]), w_o0.dtype)], axis=0)
    w0f = jnp.concatenate([w_o0[:64], jnp.take(w_cat, _SEL, axis=0)], axis=0)

    return _tc_forward(
        dense, pooled, parity,
        w_d0, b_d0[None, :], w_d1, b_d1[None, :], w_d2, b_d2[None, :],
        w0f, b_o0[None, :], w_o1, b_o1[None, :], w_o2, b_o2[None, :],
        w_o3, b_o3[None, :],
    )


# speedup vs baseline: 2.1086x; 1.6570x over previous
import numpy as np
import jax
import jax.numpy as jnp
from jax import lax
from jax.experimental import pallas as pl
from jax.experimental.pallas import tpu as pltpu
from jax.experimental.pallas import tpu_sc as plsc

B = 4096
D_IN = 13
NT = 26
V = 100000
ED = 64
NP1 = NT + 1

_NC = 2
_NS = 16
_NW = _NC * _NS
_ROWS = B * NT
_RPW = _ROWS // _NW
_CHUNK = 128
_NCHUNK = _RPW // _CHUNK
_HALF = _NCHUNK // 2

_BBLK = 128
_XW = 64 + NP1 * NP1


_BPW = B // _NW
_SB = 8
_NSUB = _BPW // _SB
_SROWS = _SB * NT


def _sc_gather_body(tbl, idxt_hbm, out_hbm, idx_v, buf_a, buf_b,
                    sem_a, sem_b):
    wid = lax.axis_index("s") * _NC + lax.axis_index("c")
    b_base = wid * _BPW
    bufs = (buf_a, buf_b)
    sems = (sem_a, sem_b)
    pend = []

    def make_issue(s, buf, sem):
        def issue(g, carry):
            i0 = g * 16
            v16 = idx_v[pl.ds(s * _SROWS + i0, 16)]
            for k in range(16):
                i = i0 + k
                bl = i // NT
                t = i % NT
                v = v16[k]
                pltpu.async_copy(tbl.at[t, pl.ds(v, 1), :],
                                 buf.at[bl, pl.ds(t, 1)], sem)
            return carry
        return issue

    pltpu.sync_copy(idxt_hbm.at[pl.ds(b_base * NT, _RPW)], idx_v)
    for s in range(_NSUB):
        buf, sem = bufs[s % 2], sems[s % 2]
        b0 = b_base + s * _SB
        lax.fori_loop(0, _SROWS // 16, make_issue(s, buf, sem), 0,
                      unroll=False)
        if pend:
            pb0, pbuf, psem = pend.pop()
            out_slc = out_hbm.at[pl.ds(pb0, _SB)]
            pltpu.make_async_copy(out_slc, pbuf, psem).wait()
            pltpu.sync_copy(pbuf, out_slc)
        pend.append((b0, buf, sem))
    pb0, pbuf, psem = pend.pop()
    out_slc = out_hbm.at[pl.ds(pb0, _SB)]
    pltpu.make_async_copy(out_slc, pbuf, psem).wait()
    pltpu.sync_copy(pbuf, out_slc)


_SC_CACHE = {}


def _sc_gather(tbl, idxt):
    if "k" not in _SC_CACHE:
        _SC_CACHE["k"] = pl.kernel(
            _sc_gather_body,
            out_type=jax.ShapeDtypeStruct((B, NT, ED), jnp.float32),
            mesh=plsc.VectorSubcoreMesh(core_axis_name="c", subcore_axis_name="s"),
            scratch_types=[
                pltpu.VMEM((_RPW,), jnp.int32),
                pltpu.VMEM((_SB, NT, ED), jnp.float32),
                pltpu.VMEM((_SB, NT, ED), jnp.float32),
                pltpu.SemaphoreType.DMA,
                pltpu.SemaphoreType.DMA,
            ],
        )
    return _SC_CACHE["k"](tbl, idxt)


def _tc_body(dense_ref, pooled_ref, wd0, bd0, wd1, bd1, wd2, bd2,
             w0f, bo0, wo1, bo1, wo2, bo2, wo3, bo3, out_ref):
    f32 = jnp.float32
    h = dense_ref[...]
    h = jnp.maximum(jnp.dot(h, wd0[...], preferred_element_type=f32) + bd0[...], 0.0)
    h = jnp.maximum(jnp.dot(h, wd1[...], preferred_element_type=f32) + bd1[...], 0.0)
    h = jnp.maximum(jnp.dot(h, wd2[...], preferred_element_type=f32) + bd2[...], 0.0)
    embs = jnp.concatenate([h[:, None, :], pooled_ref[...]], axis=1)
    inter = jax.lax.dot_general(
        embs, embs, (((2,), (2,)), ((0,), (0,))),
        preferred_element_type=f32)
    x = jnp.concatenate([h, inter.reshape(_BBLK, NP1 * NP1)], axis=1)
    x = jnp.maximum(jnp.dot(x, w0f[...], preferred_element_type=f32) + bo0[...], 0.0)
    x = jnp.maximum(jnp.dot(x, wo1[...], preferred_element_type=f32) + bo1[...], 0.0)
    x = jnp.maximum(jnp.dot(x, wo2[...], preferred_element_type=f32) + bo2[...], 0.0)
    out_ref[...] = jnp.dot(x, wo3[...], preferred_element_type=f32) + bo3[...]


_SEL = np.full((NP1 * NP1,), 442, dtype=np.int32)
_p = 0
for _n in range(NP1):
    for _m in range(_n, NP1):
        _SEL[_n * NP1 + _m] = 64 + _p
        _p += 1


def _rep(shape):
    nd = len(shape)
    return pl.BlockSpec(shape, lambda i, _nd=nd: (0,) * _nd)


def _tc_forward(dense, pooled, wd0, bd0, wd1, bd1, wd2, bd2,
                w0f, bo0, wo1, bo1, wo2, bo2, wo3, bo3):
    grid = (B // _BBLK,)
    in_specs = [
        pl.BlockSpec((_BBLK, D_IN), lambda i: (i, 0)),
        pl.BlockSpec((_BBLK, NT, ED), lambda i: (i, 0, 0)),
        _rep(wd0.shape), _rep(bd0.shape), _rep(wd1.shape), _rep(bd1.shape),
        _rep(wd2.shape), _rep(bd2.shape), _rep(w0f.shape), _rep(bo0.shape),
        _rep(wo1.shape), _rep(bo1.shape), _rep(wo2.shape), _rep(bo2.shape),
        _rep(wo3.shape), _rep(bo3.shape),
    ]
    return pl.pallas_call(
        _tc_body,
        grid=grid,
        in_specs=in_specs,
        out_specs=pl.BlockSpec((_BBLK, 1), lambda i: (i, 0)),
        out_shape=jax.ShapeDtypeStruct((B, 1), jnp.float32),
    )(dense, pooled, wd0, bd0, wd1, bd1, wd2, bd2,
      w0f, bo0, wo1, bo1, wo2, bo2, wo3, bo3)


def kernel(dense, sparse_indices, sparse_offsets, tables,
           w_d0, b_d0, w_d1, b_d1, w_d2, b_d2,
           w_o0, b_o0, w_o1, b_o1, w_o2, b_o2, w_o3, b_o3):
    del sparse_offsets
    idxt = sparse_indices.T.reshape(B * NT)
    pooled = _sc_gather(tables, idxt)

    w_cat = jnp.concatenate([w_o0, jnp.zeros((1, w_o0.shape[1]), w_o0.dtype)], axis=0)
    w0f = jnp.concatenate([w_o0[:64], jnp.take(w_cat, _SEL, axis=0)], axis=0)

    return _tc_forward(
        dense, pooled,
        w_d0, b_d0[None, :], w_d1, b_d1[None, :], w_d2, b_d2[None, :],
        w0f, b_o0[None, :], w_o1, b_o1[None, :], w_o2, b_o2[None, :],
        w_o3, b_o3[None, :],
    )


# speedup vs baseline: 3.4659x; 1.6437x over previous
import numpy as np
import jax
import jax.numpy as jnp
from jax import lax
from jax.experimental import pallas as pl
from jax.experimental.pallas import tpu as pltpu
from jax.experimental.pallas import tpu_sc as plsc

B = 4096
D_IN = 13
NT = 26
V = 100000
ED = 64
NP1 = NT + 1

_NC = 2
_NS = 16
_NW = _NC * _NS
_ROWS = B * NT
_RPW = _ROWS // _NW
_CHUNK = 128
_NCHUNK = _RPW // _CHUNK
_HALF = _NCHUNK // 2

_BBLK = 128
_XW = 64 + NP1 * NP1


_BPW = B // _NW
_SB = 8
_NSUB = _BPW // _SB
_SROWS = _SB * NT


def _sc_gather_body(tbl, idxt_hbm, out_hbm, idx_v, buf_a, buf_b,
                    sem_a, sem_b):
    wid = lax.axis_index("s") * _NC + lax.axis_index("c")
    b_base = wid * _BPW
    bufs = (buf_a, buf_b)
    sems = (sem_a, sem_b)
    pend = []

    def make_issue(s, buf, sem):
        def issue(g, carry):
            i0 = g * 16
            v16 = idx_v[pl.ds(s * _SROWS + i0, 16)]
            for k in range(16):
                i = i0 + k
                bl = i // NT
                t = i % NT
                v = v16[k]
                pltpu.async_copy(tbl.at[t // 13, t % 13, pl.ds(v, 1), :],
                                 buf.at[bl, pl.ds(t, 1)], sem)
            return carry
        return issue

    pltpu.sync_copy(idxt_hbm.at[pl.ds(b_base * NT, _RPW)], idx_v)
    for s in range(_NSUB):
        buf, sem = bufs[s % 2], sems[s % 2]
        b0 = b_base + s * _SB
        lax.fori_loop(0, _SROWS // 16, make_issue(s, buf, sem), 0,
                      unroll=False)
        if pend:
            pb0, pbuf, psem = pend.pop()
            out_slc = out_hbm.at[pl.ds(pb0, _SB)]
            pltpu.make_async_copy(out_slc, pbuf, psem).wait()
            pltpu.sync_copy(pbuf, out_slc)
        pend.append((b0, buf, sem))
    pb0, pbuf, psem = pend.pop()
    out_slc = out_hbm.at[pl.ds(pb0, _SB)]
    pltpu.make_async_copy(out_slc, pbuf, psem).wait()
    pltpu.sync_copy(pbuf, out_slc)


_SC_CACHE = {}


def _sc_gather(tbl, idxt):
    if "k" not in _SC_CACHE:
        _SC_CACHE["k"] = pl.kernel(
            _sc_gather_body,
            out_type=jax.ShapeDtypeStruct((B, NT, ED), jnp.float32),
            mesh=plsc.VectorSubcoreMesh(core_axis_name="c", subcore_axis_name="s"),
            scratch_types=[
                pltpu.VMEM((_RPW,), jnp.int32),
                pltpu.VMEM((_SB, NT, ED), jnp.float32),
                pltpu.VMEM((_SB, NT, ED), jnp.float32),
                pltpu.SemaphoreType.DMA,
                pltpu.SemaphoreType.DMA,
            ],
        )
    return _SC_CACHE["k"](tbl, idxt)


def _tc_body(dense_ref, pooled_ref, wd0, bd0, wd1, bd1, wd2, bd2,
             w0f, bo0, wo1, bo1, wo2, bo2, wo3, bo3, out_ref):
    f32 = jnp.float32
    h = dense_ref[...]
    h = jnp.maximum(jnp.dot(h, wd0[...], preferred_element_type=f32) + bd0[...], 0.0)
    h = jnp.maximum(jnp.dot(h, wd1[...], preferred_element_type=f32) + bd1[...], 0.0)
    h = jnp.maximum(jnp.dot(h, wd2[...], preferred_element_type=f32) + bd2[...], 0.0)
    embs = jnp.concatenate([h[:, None, :], pooled_ref[...]], axis=1)
    inter = jax.lax.dot_general(
        embs, embs, (((2,), (2,)), ((0,), (0,))),
        preferred_element_type=f32)
    x = jnp.concatenate([h, inter.reshape(_BBLK, NP1 * NP1)], axis=1)
    x = jnp.maximum(jnp.dot(x, w0f[...], preferred_element_type=f32) + bo0[...], 0.0)
    x = jnp.maximum(jnp.dot(x, wo1[...], preferred_element_type=f32) + bo1[...], 0.0)
    x = jnp.maximum(jnp.dot(x, wo2[...], preferred_element_type=f32) + bo2[...], 0.0)
    out_ref[...] = jnp.dot(x, wo3[...], preferred_element_type=f32) + bo3[...]


_SEL = np.full((NP1 * NP1,), 442, dtype=np.int32)
_p = 0
for _n in range(NP1):
    for _m in range(_n, NP1):
        _SEL[_n * NP1 + _m] = 64 + _p
        _p += 1


def _rep(shape):
    nd = len(shape)
    return pl.BlockSpec(shape, lambda i, _nd=nd: (0,) * _nd)


def _tc_forward(dense, pooled, wd0, bd0, wd1, bd1, wd2, bd2,
                w0f, bo0, wo1, bo1, wo2, bo2, wo3, bo3):
    grid = (B // _BBLK,)
    in_specs = [
        pl.BlockSpec((_BBLK, D_IN), lambda i: (i, 0)),
        pl.BlockSpec((_BBLK, NT, ED), lambda i: (i, 0, 0)),
        _rep(wd0.shape), _rep(bd0.shape), _rep(wd1.shape), _rep(bd1.shape),
        _rep(wd2.shape), _rep(bd2.shape), _rep(w0f.shape), _rep(bo0.shape),
        _rep(wo1.shape), _rep(bo1.shape), _rep(wo2.shape), _rep(bo2.shape),
        _rep(wo3.shape), _rep(bo3.shape),
    ]
    return pl.pallas_call(
        _tc_body,
        grid=grid,
        in_specs=in_specs,
        out_specs=pl.BlockSpec((_BBLK, 1), lambda i: (i, 0)),
        out_shape=jax.ShapeDtypeStruct((B, 1), jnp.float32),
    )(dense, pooled, wd0, bd0, wd1, bd1, wd2, bd2,
      w0f, bo0, wo1, bo1, wo2, bo2, wo3, bo3)


def kernel(dense, sparse_indices, sparse_offsets, tables,
           w_d0, b_d0, w_d1, b_d1, w_d2, b_d2,
           w_o0, b_o0, w_o1, b_o1, w_o2, b_o2, w_o3, b_o3):
    del sparse_offsets
    idxt = sparse_indices.T.reshape(B * NT)
    pooled = _sc_gather(tables.reshape(2, 13, V, ED), idxt)

    w_cat = jnp.concatenate([w_o0, jnp.zeros((1, w_o0.shape[1]), w_o0.dtype)], axis=0)
    w0f = jnp.concatenate([w_o0[:64], jnp.take(w_cat, _SEL, axis=0)], axis=0)

    return _tc_forward(
        dense, pooled,
        w_d0, b_d0[None, :], w_d1, b_d1[None, :], w_d2, b_d2[None, :],
        w0f, b_o0[None, :], w_o1, b_o1[None, :], w_o2, b_o2[None, :],
        w_o3, b_o3[None, :],
    )


# speedup vs baseline: 3.4664x; 1.0002x over previous
import numpy as np
import jax
import jax.numpy as jnp
from jax import lax
from jax.experimental import pallas as pl
from jax.experimental.pallas import tpu as pltpu
from jax.experimental.pallas import tpu_sc as plsc

B = 4096
D_IN = 13
NT = 26
V = 100000
ED = 64
NP1 = NT + 1

_NC = 2
_NS = 16
_NW = _NC * _NS
_ROWS = B * NT
_RPW = _ROWS // _NW
_CHUNK = 128
_NCHUNK = _RPW // _CHUNK
_HALF = _NCHUNK // 2

_BBLK = 128
_XW = 64 + NP1 * NP1


_BPW = B // _NW
_SB = 8
_NSUB = _BPW // _SB
_SROWS = _SB * NT


def _sc_gather_body(tbl, idxt_hbm, out_hbm, idx_v, buf_a, buf_b,
                    sem_a, sem_b):
    wid = lax.axis_index("s") * _NC + lax.axis_index("c")
    b_base = wid * _BPW
    bufs = (buf_a, buf_b)
    sems = (sem_a, sem_b)
    pend = []

    def make_issue(s, buf, sem):
        def issue(g, carry):
            i0 = g * 16
            v16 = idx_v[pl.ds(s * _SROWS + i0, 16)]
            for k in range(16):
                i = i0 + k
                bl = i // NT
                t = i % NT
                v = v16[k]
                pltpu.async_copy(tbl.at[t // 13, t % 13, pl.ds(v, 1), :],
                                 buf.at[bl, pl.ds(t, 1)], sem)
            return carry
        return issue

    pltpu.sync_copy(idxt_hbm.at[pl.ds(b_base * NT, _RPW)], idx_v)
    for s in range(_NSUB):
        buf, sem = bufs[s % 2], sems[s % 2]
        b0 = b_base + s * _SB
        lax.fori_loop(0, _SROWS // 16, make_issue(s, buf, sem), 0,
                      unroll=False)
        if pend:
            pb0, pbuf, psem = pend.pop()
            out_slc = out_hbm.at[pl.ds(pb0, _SB)]
            pltpu.make_async_copy(out_slc, pbuf, psem).wait()
            pltpu.sync_copy(pbuf, out_slc)
        pend.append((b0, buf, sem))
    pb0, pbuf, psem = pend.pop()
    out_slc = out_hbm.at[pl.ds(pb0, _SB)]
    pltpu.make_async_copy(out_slc, pbuf, psem).wait()
    pltpu.sync_copy(pbuf, out_slc)


_SC_CACHE = {}


def _sc_gather(tbl, idxt):
    if "k" not in _SC_CACHE:
        _SC_CACHE["k"] = pl.kernel(
            _sc_gather_body,
            out_type=jax.ShapeDtypeStruct((B, NT, ED), jnp.float32),
            mesh=plsc.VectorSubcoreMesh(core_axis_name="c", subcore_axis_name="s"),
            scratch_types=[
                pltpu.VMEM((_RPW,), jnp.int32),
                pltpu.VMEM((_SB, NT, ED), jnp.float32),
                pltpu.VMEM((_SB, NT, ED), jnp.float32),
                pltpu.SemaphoreType.DMA,
                pltpu.SemaphoreType.DMA,
            ],
        )
    return _SC_CACHE["k"](tbl, idxt)


def _tc_body(dense_ref, pooled_ref, wd0, bd0, wd1, bd1, wd2, bd2,
             w0f, bo0, wo1, bo1, wo2, bo2, wo3, bo3, out_ref):
    f32 = jnp.float32
    h = dense_ref[...]
    h = jnp.maximum(jnp.dot(h, wd0[...], preferred_element_type=f32) + bd0[...], 0.0)
    h = jnp.maximum(jnp.dot(h, wd1[...], preferred_element_type=f32) + bd1[...], 0.0)
    h = jnp.maximum(jnp.dot(h, wd2[...], preferred_element_type=f32) + bd2[...], 0.0)
    embs = jnp.concatenate([h[:, None, :], pooled_ref[...]], axis=1)
    inter = jax.lax.dot_general(
        embs, embs, (((2,), (2,)), ((0,), (0,))),
        preferred_element_type=f32)
    x = jnp.concatenate([h, inter.reshape(_BBLK, NP1 * NP1)], axis=1)
    x = jnp.maximum(jnp.dot(x, w0f[...], preferred_element_type=f32) + bo0[...], 0.0)
    x = jnp.maximum(jnp.dot(x, wo1[...], preferred_element_type=f32) + bo1[...], 0.0)
    x = jnp.maximum(jnp.dot(x, wo2[...], preferred_element_type=f32) + bo2[...], 0.0)
    out_ref[...] = jnp.dot(x, wo3[...], preferred_element_type=f32) + bo3[...]


_SEL = np.full((NP1 * NP1,), 442, dtype=np.int32)
_p = 0
for _n in range(NP1):
    for _m in range(_n, NP1):
        _SEL[_n * NP1 + _m] = 64 + _p
        _p += 1


def _rep(shape):
    nd = len(shape)
    return pl.BlockSpec(shape, lambda i, _nd=nd: (0,) * _nd)


def _tc_forward(dense, pooled, wd0, bd0, wd1, bd1, wd2, bd2,
                w0f, bo0, wo1, bo1, wo2, bo2, wo3, bo3):
    grid = (B // _BBLK,)
    in_specs = [
        pl.BlockSpec((_BBLK, D_IN), lambda i: (i, 0)),
        pl.BlockSpec((_BBLK, NT, ED), lambda i: (i, 0, 0)),
        _rep(wd0.shape), _rep(bd0.shape), _rep(wd1.shape), _rep(bd1.shape),
        _rep(wd2.shape), _rep(bd2.shape), _rep(w0f.shape), _rep(bo0.shape),
        _rep(wo1.shape), _rep(bo1.shape), _rep(wo2.shape), _rep(bo2.shape),
        _rep(wo3.shape), _rep(bo3.shape),
    ]
    return pl.pallas_call(
        _tc_body,
        grid=grid,
        in_specs=in_specs,
        out_specs=pl.BlockSpec((_BBLK, 1), lambda i: (i, 0)),
        out_shape=jax.ShapeDtypeStruct((B, 1), jnp.float32),
        compiler_params=pltpu.CompilerParams(
            dimension_semantics=("parallel",)),
    )(dense, pooled, wd0, bd0, wd1, bd1, wd2, bd2,
      w0f, bo0, wo1, bo1, wo2, bo2, wo3, bo3)


def kernel(dense, sparse_indices, sparse_offsets, tables,
           w_d0, b_d0, w_d1, b_d1, w_d2, b_d2,
           w_o0, b_o0, w_o1, b_o1, w_o2, b_o2, w_o3, b_o3):
    del sparse_offsets
    idxt = sparse_indices.T.reshape(B * NT)
    pooled = _sc_gather(tables.reshape(2, 13, V, ED), idxt)

    w_cat = jnp.concatenate([w_o0, jnp.zeros((1, w_o0.shape[1]), w_o0.dtype)], axis=0)
    w0f = jnp.concatenate([w_o0[:64], jnp.take(w_cat, _SEL, axis=0)], axis=0)

    return _tc_forward(
        dense, pooled,
        w_d0, b_d0[None, :], w_d1, b_d1[None, :], w_d2, b_d2[None, :],
        w0f, b_o0[None, :], w_o1, b_o1[None, :], w_o2, b_o2[None, :],
        w_o3, b_o3[None, :],
    )


# speedup vs baseline: 3.4845x; 1.0052x over previous
import numpy as np
import jax
import jax.numpy as jnp
from jax import lax
from jax.experimental import pallas as pl
from jax.experimental.pallas import tpu as pltpu
from jax.experimental.pallas import tpu_sc as plsc

B = 4096
D_IN = 13
NT = 26
V = 100000
ED = 64
NP1 = NT + 1

_NC = 2
_NS = 16
_NW = _NC * _NS
_ROWS = B * NT
_RPW = _ROWS // _NW
_CHUNK = 128
_NCHUNK = _RPW // _CHUNK
_HALF = _NCHUNK // 2

_BBLK = 128
_XW = 64 + NP1 * NP1


_BPW = B // _NW
_SB = 8
_NSUB = _BPW // _SB
_SROWS = _SB * NT


def _sc_gather_body(tbl, idxt_hbm, out_hbm, idx_v, buf_a, buf_b,
                    sem_a, sem_b):
    wid = lax.axis_index("s") * _NC + lax.axis_index("c")
    b_base = wid * _BPW
    bufs = (buf_a, buf_b)
    sems = (sem_a, sem_b)
    pend = []

    def make_issue(s, buf, sem):
        def issue(g, carry):
            i0 = g * 16
            v16 = idx_v[pl.ds(s * _SROWS + i0, 16)]
            for k in range(16):
                i = i0 + k
                bl = i // NT
                t = i % NT
                v = v16[k]
                pltpu.async_copy(tbl.at[t // 13, t % 13, pl.ds(v, 1), :],
                                 buf.at[bl, pl.ds(t, 1)], sem)
            return carry
        return issue

    pltpu.sync_copy(idxt_hbm.at[pl.ds(b_base * NT, _RPW)], idx_v)
    for s in range(_NSUB):
        buf, sem = bufs[s % 2], sems[s % 2]
        b0 = b_base + s * _SB
        lax.fori_loop(0, _SROWS // 16, make_issue(s, buf, sem), 0,
                      unroll=False)
        if pend:
            pb0, pbuf, psem = pend.pop()
            out_slc = out_hbm.at[pl.ds(pb0, _SB)]
            pltpu.make_async_copy(out_slc, pbuf, psem).wait()
            pltpu.sync_copy(pbuf, out_slc)
        pend.append((b0, buf, sem))
    pb0, pbuf, psem = pend.pop()
    out_slc = out_hbm.at[pl.ds(pb0, _SB)]
    pltpu.make_async_copy(out_slc, pbuf, psem).wait()
    pltpu.sync_copy(pbuf, out_slc)


_SC_CACHE = {}


def _sc_gather(tbl, idxt):
    if "k" not in _SC_CACHE:
        _SC_CACHE["k"] = pl.kernel(
            _sc_gather_body,
            out_type=jax.ShapeDtypeStruct((B, NT, ED), jnp.float32),
            mesh=plsc.VectorSubcoreMesh(core_axis_name="c", subcore_axis_name="s"),
            scratch_types=[
                pltpu.VMEM((_RPW,), jnp.int32),
                pltpu.VMEM((_SB, NT, ED), jnp.float32),
                pltpu.VMEM((_SB, NT, ED), jnp.float32),
                pltpu.SemaphoreType.DMA,
                pltpu.SemaphoreType.DMA,
            ],
        )
    return _SC_CACHE["k"](tbl, idxt)


def _tc_body(dense_ref, pooled_ref, wd0, bd0, wd1, bd1, wd2, bd2,
             wh, wpp, bo0, wo1, bo1, wo2, bo2, wo3, bo3, out_ref):
    f32 = jnp.float32
    h = dense_ref[...]
    h = jnp.maximum(jnp.dot(h, wd0[...], preferred_element_type=f32) + bd0[...], 0.0)
    h = jnp.maximum(jnp.dot(h, wd1[...], preferred_element_type=f32) + bd1[...], 0.0)
    h = jnp.maximum(jnp.dot(h, wd2[...], preferred_element_type=f32) + bd2[...], 0.0)
    embs = jnp.concatenate([h[:, None, :], pooled_ref[...]], axis=1)
    inter = jax.lax.dot_general(
        embs, embs, (((2,), (2,)), ((0,), (0,))),
        preferred_element_type=f32)
    x = (jnp.dot(h, wh[...], preferred_element_type=f32)
         + jnp.dot(inter.reshape(_BBLK, NP1 * NP1), wpp[...],
                   preferred_element_type=f32))
    x = jnp.maximum(x + bo0[...], 0.0)
    x = jnp.maximum(jnp.dot(x, wo1[...], preferred_element_type=f32) + bo1[...], 0.0)
    x = jnp.maximum(jnp.dot(x, wo2[...], preferred_element_type=f32) + bo2[...], 0.0)
    out_ref[...] = jnp.dot(x, wo3[...], preferred_element_type=f32) + bo3[...]


_SEL = np.full((NP1 * NP1,), 442, dtype=np.int32)
_p = 0
for _n in range(NP1):
    for _m in range(_n, NP1):
        _SEL[_n * NP1 + _m] = 64 + _p
        _p += 1


def _rep(shape):
    nd = len(shape)
    return pl.BlockSpec(shape, lambda i, _nd=nd: (0,) * _nd)


def _tc_forward(dense, pooled, wd0, bd0, wd1, bd1, wd2, bd2,
                wh, wpp, bo0, wo1, bo1, wo2, bo2, wo3, bo3):
    grid = (B // _BBLK,)
    in_specs = [
        pl.BlockSpec((_BBLK, D_IN), lambda i: (i, 0)),
        pl.BlockSpec((_BBLK, NT, ED), lambda i: (i, 0, 0)),
        _rep(wd0.shape), _rep(bd0.shape), _rep(wd1.shape), _rep(bd1.shape),
        _rep(wd2.shape), _rep(bd2.shape), _rep(wh.shape),
        _rep(wpp.shape), _rep(bo0.shape),
        _rep(wo1.shape), _rep(bo1.shape), _rep(wo2.shape), _rep(bo2.shape),
        _rep(wo3.shape), _rep(bo3.shape),
    ]
    return pl.pallas_call(
        _tc_body,
        grid=grid,
        in_specs=in_specs,
        out_specs=pl.BlockSpec((_BBLK, 1), lambda i: (i, 0)),
        out_shape=jax.ShapeDtypeStruct((B, 1), jnp.float32),
        compiler_params=pltpu.CompilerParams(
            dimension_semantics=("parallel",)),
    )(dense, pooled, wd0, bd0, wd1, bd1, wd2, bd2,
      wh, wpp, bo0, wo1, bo1, wo2, bo2, wo3, bo3)


def kernel(dense, sparse_indices, sparse_offsets, tables,
           w_d0, b_d0, w_d1, b_d1, w_d2, b_d2,
           w_o0, b_o0, w_o1, b_o1, w_o2, b_o2, w_o3, b_o3):
    del sparse_offsets
    idxt = sparse_indices.T.reshape(B * NT)
    pooled = _sc_gather(tables.reshape(2, 13, V, ED), idxt)

    w_cat = jnp.concatenate([w_o0, jnp.zeros((1, w_o0.shape[1]), w_o0.dtype)], axis=0)
    wh = w_o0[:64]
    wpp = jnp.take(w_cat, _SEL, axis=0)

    return _tc_forward(
        dense, pooled,
        w_d0, b_d0[None, :], w_d1, b_d1[None, :], w_d2, b_d2[None, :],
        wh, wpp, b_o0[None, :], w_o1, b_o1[None, :], w_o2, b_o2[None, :],
        w_o3, b_o3[None, :],
    )
